# Initial kernel scaffold; baseline (speedup 1.0000x reference)
import jax
import jax.numpy as jnp
from jax.experimental import pallas as pl


def kernel(h, edge_index, edge_norm, W, b):
    raise NotImplementedError("write your pallas kernel here")


# speedup vs baseline: 4.5397x; 4.5397x over previous
import functools

import jax
import jax.numpy as jnp
from jax import lax
from jax.experimental import pallas as pl
from jax.experimental.pallas import tpu as pltpu
from jax.experimental.pallas import tpu_sc as plsc

N_NODES = 10000
N_EDGES = 160000
D_IN = 256
D_OUT = 256

NC = 2
NS = 16
K = 128
NWIN = 80
E_PAD = NS * NWIN * K
EDGES_PER_SUBCORE = NWIN * K
HALF = 128
N_PAD = 10240
ROWS_PER_SUBCORE = N_PAD // NS


def _sc_agg(srcidx, dst2, norm2, h2):
    mesh = plsc.VectorSubcoreMesh(core_axis_name="c", subcore_axis_name="s")

    @functools.partial(
        pl.kernel,
        out_type=jax.ShapeDtypeStruct((NC, N_NODES, HALF), jnp.float32),
        mesh=mesh,
        scratch_types=[
            pltpu.VMEM((NWIN, K), jnp.int32),
            pltpu.VMEM((NWIN, K), jnp.int32),
            pltpu.VMEM((NWIN, K), jnp.float32),
            pltpu.VMEM((K, HALF), jnp.float32),
            pltpu.VMEM_SHARED((N_PAD, HALF), jnp.float32),
        ],
    )
    def sc_kernel(srcidx_hbm, dst_hbm, norm_hbm, h2_hbm, out_hbm,
                  src_v, dst_v, norm_v, rows_v, acc):
        c = lax.axis_index("c")
        s = lax.axis_index("s")
        base = s * NWIN

        pltpu.sync_copy(srcidx_hbm.at[c].at[pl.ds(base, NWIN)], src_v)
        pltpu.sync_copy(dst_hbm.at[pl.ds(base, NWIN)], dst_v)
        pltpu.sync_copy(norm_hbm.at[pl.ds(base, NWIN)], norm_v)

        @pl.loop(0, K)
        def _(r):
            for j in range(HALF // 16):
                rows_v[r, pl.ds(j * 16, 16)] = jnp.zeros((16,), jnp.float32)

        row0 = s * ROWS_PER_SUBCORE
        for i in range(ROWS_PER_SUBCORE // K):
            pltpu.sync_copy(rows_v.at[pl.ds(0, K)],
                            acc.at[pl.ds(row0 + i * K, K)])

        plsc.subcore_barrier()

        @pl.loop(0, NWIN)
        def _(w):
            pltpu.sync_copy(h2_hbm.at[src_v.at[w]], rows_v)

            @pl.loop(0, K, step=16)
            def _(g):
                nv = norm_v[w, pl.ds(g, 16)]
                for i in range(16):
                    t = nv[i]
                    for j in range(HALF // 16):
                        sl = pl.ds(j * 16, 16)
                        rows_v[g + i, sl] = rows_v[g + i, sl] * t

            pltpu.sync_copy(rows_v, acc.at[dst_v.at[w]], add=True)

        plsc.subcore_barrier()

        last_rows = N_NODES - (NS - 1) * ROWS_PER_SUBCORE

        @pl.when(s < NS - 1)
        def _():
            pltpu.sync_copy(acc.at[pl.ds(row0, ROWS_PER_SUBCORE)],
                            out_hbm.at[c].at[pl.ds(row0, ROWS_PER_SUBCORE)])

        @pl.when(s == NS - 1)
        def _():
            pltpu.sync_copy(acc.at[pl.ds(row0, last_rows)],
                            out_hbm.at[c].at[pl.ds(row0, last_rows)])

    return sc_kernel(srcidx, dst2, norm2, h2)


def _tc_matmul_body(a0_ref, a1_ref, w_ref, b_ref, o_ref):
    dn = (((1,), (1,)), ((), ()))
    acc = lax.dot_general(a0_ref[...], w_ref[:, 0:HALF], dn,
                          preferred_element_type=jnp.float32)
    acc = acc + lax.dot_general(a1_ref[...], w_ref[:, HALF:D_IN], dn,
                                preferred_element_type=jnp.float32)
    o_ref[...] = acc + b_ref[...]


def _tc_matmul(a0, a1, W, b2d):
    blk = 1000
    grid = (N_NODES // blk,)
    return pl.pallas_call(
        _tc_matmul_body,
        grid=grid,
        in_specs=[
            pl.BlockSpec((blk, HALF), lambda i: (i, 0)),
            pl.BlockSpec((blk, HALF), lambda i: (i, 0)),
            pl.BlockSpec((D_OUT, D_IN), lambda i: (0, 0)),
            pl.BlockSpec((1, D_OUT), lambda i: (0, 0)),
        ],
        out_specs=pl.BlockSpec((blk, D_OUT), lambda i: (i, 0)),
        out_shape=jax.ShapeDtypeStruct((N_NODES, D_OUT), jnp.float32),
    )(a0, a1, W, b2d)


def kernel(h, edge_index, edge_norm, W, b):
    src = edge_index[0].astype(jnp.int32)
    dst = edge_index[1].astype(jnp.int32)
    norm = edge_norm.reshape(-1).astype(jnp.float32)

    pad = E_PAD - N_EDGES
    fill = (jnp.arange(pad, dtype=jnp.int32) * 7919) % N_NODES
    src_p = jnp.concatenate([src, fill])
    dst_p = jnp.concatenate([dst, fill])
    norm_p = jnp.concatenate([norm, jnp.zeros((pad,), jnp.float32)])

    srcidx = (src_p * 2)[None, :] + jnp.arange(NC, dtype=jnp.int32)[:, None]
    srcidx = srcidx.reshape(NC, NS * NWIN, K)
    dst2 = dst_p.reshape(NS * NWIN, K)
    norm2 = norm_p.reshape(NS * NWIN, K)
    h2 = h.reshape(2 * N_NODES, HALF)

    acc = _sc_agg(srcidx, dst2, norm2, h2)
    return _tc_matmul(acc[0], acc[1], W, b.reshape(1, D_OUT))


# speedup vs baseline: 6.8385x; 1.5064x over previous
import functools

import jax
import jax.numpy as jnp
from jax import lax
from jax.experimental import pallas as pl
from jax.experimental.pallas import tpu as pltpu
from jax.experimental.pallas import tpu_sc as plsc

N_NODES = 10000
N_EDGES = 160000
D_IN = 256
D_OUT = 256

NC = 2
NS = 16
K = 128
NWIN = 80
E_PAD = NS * NWIN * K
EDGES_PER_SUBCORE = NWIN * K
HALF = 128
N_PAD = 10240
ROWS_PER_SUBCORE = N_PAD // NS


def _sc_agg(packed2, norm2, h2):
    mesh = plsc.VectorSubcoreMesh(core_axis_name="c", subcore_axis_name="s")

    @functools.partial(
        pl.kernel,
        out_type=jax.ShapeDtypeStruct((NC, N_NODES, HALF), jnp.float32),
        mesh=mesh,
        scratch_types=[
            pltpu.VMEM((NWIN, K), jnp.int32),
            pltpu.VMEM((2, K), jnp.int32),
            pltpu.VMEM((2, K), jnp.int32),
            pltpu.VMEM((2, K), jnp.float32),
            pltpu.VMEM((2, K, HALF), jnp.float32),
            pltpu.VMEM_SHARED((N_PAD, HALF), jnp.float32),
            pltpu.SemaphoreType.DMA,
            pltpu.SemaphoreType.DMA,
            pltpu.SemaphoreType.DMA,
            pltpu.SemaphoreType.DMA,
        ],
    )
    def sc_kernel(packed_hbm, norm_hbm, h2_hbm, out_hbm,
                  pk_v, src_w, dst_w, norm_w, rows_v, acc,
                  gsem0, gsem1, nsem0, nsem1):
        c = lax.axis_index("c")
        s = lax.axis_index("s")
        base = s * NWIN

        pltpu.sync_copy(packed_hbm.at[pl.ds(base, NWIN)], pk_v)

        @pl.loop(0, K)
        def _(r):
            for j in range(HALF // 16):
                rows_v[0, r, pl.ds(j * 16, 16)] = jnp.zeros((16,), jnp.float32)

        row0 = s * ROWS_PER_SUBCORE
        for i in range(ROWS_PER_SUBCORE // K):
            pltpu.sync_copy(rows_v.at[0],
                            acc.at[pl.ds(row0 + i * K, K)])

        plsc.subcore_barrier()

        gsems = (gsem0, gsem1)
        nsems = (nsem0, nsem1)

        def unpack(w, p):
            for j in range(K // 16):
                sl = pl.ds(j * 16, 16)
                v = pk_v[w, sl]
                src_w[p, sl] = ((v & 0xFFFF) << 1) + c
                dst_w[p, sl] = v >> 16

        def gather(p):
            return pltpu.make_async_copy(h2_hbm.at[src_w.at[p]],
                                         rows_v.at[p], gsems[p])

        def normcp(w, p):
            return pltpu.make_async_copy(norm_hbm.at[base + w],
                                         norm_w.at[p], nsems[p])

        def prefetch(w, p):
            unpack(w, p)
            gather(p).start()
            normcp(w, p).start()

        def scale_and_scatter(p):
            @pl.loop(0, K, step=16)
            def _(g):
                nv = norm_w[p, pl.ds(g, 16)]
                for i in range(16):
                    t = nv[i]
                    for j in range(HALF // 16):
                        sl = pl.ds(j * 16, 16)
                        rows_v[p, g + i, sl] = rows_v[p, g + i, sl] * t

            pltpu.sync_copy(rows_v.at[p], acc.at[dst_w.at[p]], add=True)

        prefetch(0, 0)
        prefetch(1, 1)

        @pl.loop(0, NWIN, step=2)
        def _(w):
            for p in range(2):
                gather(p).wait()
                normcp(w + p, p).wait()
                scale_and_scatter(p)

                @pl.when(w + 2 + p < NWIN)
                def _():
                    prefetch(w + 2 + p, p)

        plsc.subcore_barrier()

        last_rows = N_NODES - (NS - 1) * ROWS_PER_SUBCORE

        @pl.when(s < NS - 1)
        def _():
            pltpu.sync_copy(acc.at[pl.ds(row0, ROWS_PER_SUBCORE)],
                            out_hbm.at[c].at[pl.ds(row0, ROWS_PER_SUBCORE)])

        @pl.when(s == NS - 1)
        def _():
            pltpu.sync_copy(acc.at[pl.ds(row0, last_rows)],
                            out_hbm.at[c].at[pl.ds(row0, last_rows)])

    return sc_kernel(packed2, norm2, h2)


def _tc_matmul_body(a0_ref, a1_ref, w_ref, b_ref, o_ref):
    dn = (((1,), (1,)), ((), ()))
    acc = lax.dot_general(a0_ref[...], w_ref[:, 0:HALF], dn,
                          preferred_element_type=jnp.float32)
    acc = acc + lax.dot_general(a1_ref[...], w_ref[:, HALF:D_IN], dn,
                                preferred_element_type=jnp.float32)
    o_ref[...] = acc + b_ref[...]


def _tc_matmul(a0, a1, W, b2d):
    blk = 1000
    grid = (N_NODES // blk,)
    return pl.pallas_call(
        _tc_matmul_body,
        grid=grid,
        in_specs=[
            pl.BlockSpec((blk, HALF), lambda i: (i, 0)),
            pl.BlockSpec((blk, HALF), lambda i: (i, 0)),
            pl.BlockSpec((D_OUT, D_IN), lambda i: (0, 0)),
            pl.BlockSpec((1, D_OUT), lambda i: (0, 0)),
        ],
        out_specs=pl.BlockSpec((blk, D_OUT), lambda i: (i, 0)),
        out_shape=jax.ShapeDtypeStruct((N_NODES, D_OUT), jnp.float32),
    )(a0, a1, W, b2d)


def kernel(h, edge_index, edge_norm, W, b):
    src = edge_index[0].astype(jnp.int32)
    dst = edge_index[1].astype(jnp.int32)
    norm = edge_norm.reshape(-1).astype(jnp.float32)

    pad = E_PAD - N_EDGES
    fill = (jnp.arange(pad, dtype=jnp.int32) * 7919) % N_NODES
    src_p = jnp.concatenate([src, fill])
    dst_p = jnp.concatenate([dst, fill])
    norm_p = jnp.concatenate([norm, jnp.zeros((pad,), jnp.float32)])

    packed2 = (src_p | (dst_p << 16)).reshape(NS * NWIN, K)
    norm2 = norm_p.reshape(NS * NWIN, K)
    h2 = h.reshape(2 * N_NODES, HALF)

    acc = _sc_agg(packed2, norm2, h2)
    return _tc_matmul(acc[0], acc[1], W, b.reshape(1, D_OUT))
